# Initial kernel scaffold; baseline (speedup 1.0000x reference)
import jax
import jax.numpy as jnp
from jax.experimental import pallas as pl


def kernel(gate_logits, bias):
    raise NotImplementedError("write your pallas kernel here")


# speedup vs baseline: 1.1867x; 1.1867x over previous
import jax
import jax.numpy as jnp
from jax import lax
from jax.experimental import pallas as pl
from jax.experimental.pallas import tpu as pltpu
from jax.experimental.pallas import tpu_sc as plsc

E = 64
K = 8
T = 32768
NC = 2
NS = 16
NW = NC * NS
TOK = T // NW
PAIRS = TOK // 2


def _body(gate_ref, bias_ref, idx_out_ref, w_out_ref, x_v, bias_v, iout_v,
          wout_v):
  wid = lax.axis_index("c") * NS + lax.axis_index("s")
  base = wid * TOK
  pltpu.sync_copy(gate_ref.at[pl.ds(base * E, TOK * E)], x_v)
  pltpu.sync_copy(bias_ref, bias_v)

  iota = lax.iota(jnp.int32, 16)
  mask8 = iota < 8
  shift8 = (iota + 8) & 15
  full7 = jnp.full((16,), 7, jnp.int32)
  full15 = jnp.full((16,), 15, jnp.int32)
  bias_c = [bias_v[pl.ds(16 * j, 16)] for j in range(4)]
  idx_c = [iota + 16 * j for j in range(4)]

  def combine(uv, ui, vv, vi):
    vvs = jnp.take_along_axis(vv, shift8, axis=0)
    vis = jnp.take_along_axis(vi, shift8, axis=0)
    return jnp.where(mask8, uv, vvs), jnp.where(mask8, ui, vis)

  def top8(tok):
    off = tok * E
    s = []
    for j in range(4):
      c = x_v[pl.ds(off + 16 * j, 16)] + bias_c[j]
      s.append(plsc.sort_key_val(c, idx_c[j], descending=True))
    xv, xi = combine(s[0][0], s[0][1], s[1][0], s[1][1])
    yv, yi = combine(s[2][0], s[2][1], s[3][0], s[3][1])
    xv, xi = plsc.sort_key_val(xv, xi, descending=True)
    yv, yi = plsc.sort_key_val(yv, yi, descending=True)
    zv, zi = combine(xv, xi, yv, yi)
    return plsc.sort_key_val(zv, zi, descending=True)

  def pair_body(p, carry):
    av, ai = top8(2 * p)
    bv, bi = top8(2 * p + 1)
    wv, wi = combine(av, ai, bv, bi)
    e = jnp.exp(wv)
    c = plsc.cumsum(e)
    g7 = jnp.take_along_axis(c, full7, axis=0)
    g15 = jnp.take_along_axis(c, full15, axis=0)
    denom = jnp.where(mask8, g7, g15 - g7)
    iout_v[pl.ds(p * 16, 16)] = wi
    wout_v[pl.ds(p * 16, 16)] = e / denom
    return carry

  lax.fori_loop(0, PAIRS, pair_body, 0)

  pltpu.sync_copy(iout_v, idx_out_ref.at[pl.ds(base * K, TOK * K)])
  pltpu.sync_copy(wout_v, w_out_ref.at[pl.ds(base * K, TOK * K)])


_router = pl.kernel(
    _body,
    out_type=(
        jax.ShapeDtypeStruct((T * K,), jnp.int32),
        jax.ShapeDtypeStruct((T * K,), jnp.float32),
    ),
    mesh=plsc.VectorSubcoreMesh(
        core_axis_name="c", subcore_axis_name="s", num_cores=NC,
        num_subcores=NS),
    compiler_params=pltpu.CompilerParams(needs_layout_passes=False),
    scratch_types=[
        pltpu.VMEM((TOK * E,), jnp.float32),
        pltpu.VMEM((E,), jnp.float32),
        pltpu.VMEM((TOK * K,), jnp.int32),
        pltpu.VMEM((TOK * K,), jnp.float32),
    ],
)


def kernel(gate_logits, bias):
  idx_flat, w_flat = _router(gate_logits.reshape(-1), bias)
  return idx_flat.reshape(T, K), w_flat.reshape(T, K)


# speedup vs baseline: 1.4820x; 1.2489x over previous
import jax
import jax.numpy as jnp
from jax import lax
from jax.experimental import pallas as pl
from jax.experimental.pallas import tpu as pltpu
from jax.experimental.pallas import tpu_sc as plsc

E = 64
K = 8
T = 32768
NC = 2
NS = 16
NW = NC * NS
TOK = T // NW
PAIRS = TOK // 2


def _body(gate_ref, bias_ref, idx_out_ref, w_out_ref, x_v, bias_v, iout_v,
          wout_v):
  wid = lax.axis_index("c") * NS + lax.axis_index("s")
  base = wid * TOK
  pltpu.sync_copy(gate_ref.at[pl.ds(base * E, TOK * E)], x_v)
  pltpu.sync_copy(bias_ref, bias_v)

  iota = lax.iota(jnp.int32, 16)
  mask8 = iota < 8
  shift8 = (iota + 8) & 15
  full7 = jnp.full((16,), 7, jnp.int32)
  full15 = jnp.full((16,), 15, jnp.int32)
  bias_c = [bias_v[pl.ds(16 * j, 16)] for j in range(4)]
  idx_c = [iota + 16 * j for j in range(4)]

  def combine(uv, ui, vv, vi):
    vvs = jnp.take_along_axis(vv, shift8, axis=0)
    vis = jnp.take_along_axis(vi, shift8, axis=0)
    return jnp.where(mask8, uv, vvs), jnp.where(mask8, ui, vis)

  def top8(tok):
    off = tok * E
    s = []
    for j in range(4):
      c = x_v[pl.ds(off + 16 * j, 16)] + bias_c[j]
      s.append(plsc.sort_key_val(c, idx_c[j], descending=True))
    xv, xi = combine(s[0][0], s[0][1], s[1][0], s[1][1])
    yv, yi = combine(s[2][0], s[2][1], s[3][0], s[3][1])
    xv, xi = plsc.sort_key_val(xv, xi, descending=True)
    yv, yi = plsc.sort_key_val(yv, yi, descending=True)
    zv, zi = combine(xv, xi, yv, yi)
    return plsc.sort_key_val(zv, zi, descending=True)

  @plsc.parallel_loop(0, PAIRS, 1, unroll=4)
  def _pair_loop(p):
    av, ai = top8(2 * p)
    bv, bi = top8(2 * p + 1)
    wv, wi = combine(av, ai, bv, bi)
    e = jnp.exp(wv)
    c = plsc.cumsum(e)
    g7 = jnp.take_along_axis(c, full7, axis=0)
    g15 = jnp.take_along_axis(c, full15, axis=0)
    denom = jnp.where(mask8, g7, g15 - g7)
    iout_v[pl.ds(p * 16, 16)] = wi
    wout_v[pl.ds(p * 16, 16)] = e / denom

  pltpu.sync_copy(iout_v, idx_out_ref.at[pl.ds(base * K, TOK * K)])
  pltpu.sync_copy(wout_v, w_out_ref.at[pl.ds(base * K, TOK * K)])


_router = pl.kernel(
    _body,
    out_type=(
        jax.ShapeDtypeStruct((T * K,), jnp.int32),
        jax.ShapeDtypeStruct((T * K,), jnp.float32),
    ),
    mesh=plsc.VectorSubcoreMesh(
        core_axis_name="c", subcore_axis_name="s", num_cores=NC,
        num_subcores=NS),
    compiler_params=pltpu.CompilerParams(needs_layout_passes=False),
    scratch_types=[
        pltpu.VMEM((TOK * E,), jnp.float32),
        pltpu.VMEM((E,), jnp.float32),
        pltpu.VMEM((TOK * K,), jnp.int32),
        pltpu.VMEM((TOK * K,), jnp.float32),
    ],
)


def kernel(gate_logits, bias):
  idx_flat, w_flat = _router(gate_logits.reshape(-1), bias)
  return idx_flat.reshape(T, K), w_flat.reshape(T, K)
